# Initial kernel scaffold; baseline (speedup 1.0000x reference)
import jax
import jax.numpy as jnp
from jax.experimental import pallas as pl


def kernel(table, inputs):
    raise NotImplementedError("write your pallas kernel here")


# speedup vs baseline: 7.4130x; 7.4130x over previous
import functools

import jax
import jax.numpy as jnp
from jax.experimental import pallas as pl
from jax.experimental.pallas import tpu as pltpu
from jax.experimental.pallas import tpu_sc as plsc

_PAD = 0
_WINDOW = 128


def _gather_sc(table, idx2d):
    n_idx = idx2d.shape[1]
    emb = table.shape[1]
    mesh = plsc.VectorSubcoreMesh(core_axis_name="core",
                                  subcore_axis_name="subcore")

    @functools.partial(
        pl.kernel,
        out_type=jax.ShapeDtypeStruct((n_idx, emb), table.dtype),
        mesh=mesh,
    )
    def k(x_hbm, i_hbm, o_hbm):
        def body(i_vmem, o_vmem):
            pltpu.sync_copy(x_hbm.at[i_vmem.at[0]], o_vmem)

        pltpu.emit_pipeline(
            body,
            grid=(n_idx // _WINDOW,),
            in_specs=[pl.BlockSpec((1, _WINDOW), index_map=lambda i: (0, i))],
            out_specs=[pl.BlockSpec((_WINDOW, emb), index_map=lambda i: (i, 0))],
            core_axis_name=("core", "subcore"),
            dimension_semantics=(pltpu.PARALLEL,),
        )(i_hbm, o_hbm)

    return k(table, idx2d)


def _mask_len_tc(inputs):
    b, l = inputs.shape

    def body(x_ref, m_ref, len_ref):
        x = x_ref[...]
        m = x != _PAD
        m_ref[...] = m
        len_ref[...] = jnp.sum(m.astype(jnp.int32), axis=1, keepdims=True)

    return pl.pallas_call(
        body,
        out_shape=(jax.ShapeDtypeStruct((b, l), jnp.bool_),
                   jax.ShapeDtypeStruct((b, 1), jnp.int32)),
    )(inputs)


def kernel(table, inputs):
    b, l = inputs.shape
    emb = table.shape[1]
    idx2d = inputs.reshape(1, b * l)
    emb_flat = _gather_sc(table, idx2d)
    masks, lengths = _mask_len_tc(inputs)
    return emb_flat.reshape(b, l, emb), lengths.reshape(b), masks


# speedup vs baseline: 9.1127x; 1.2293x over previous
import functools

import jax
import jax.numpy as jnp
from jax.experimental import pallas as pl
from jax.experimental.pallas import tpu as pltpu
from jax.experimental.pallas import tpu_sc as plsc

_PAD = 0
_WINDOW = 128


_G = 2


def _gather_sc(table, idx2d):
    n_rows, w = idx2d.shape
    n_idx = n_rows * w
    emb = table.shape[1]
    mesh = plsc.VectorSubcoreMesh(core_axis_name="core",
                                  subcore_axis_name="subcore")

    @functools.partial(
        pl.kernel,
        out_type=jax.ShapeDtypeStruct((n_idx, emb), table.dtype),
        mesh=mesh,
        scratch_types=[pltpu.SemaphoreType.DMA],
    )
    def k(x_hbm, i_hbm, o_hbm, sem):
        def body(i_vmem, o_vmem):
            cps = [
                pltpu.async_copy(
                    x_hbm.at[i_vmem.at[j]],
                    o_vmem.at[pl.ds(j * w, w), :],
                    sem,
                )
                for j in range(_G)
            ]
            for cp in cps:
                cp.wait()

        pltpu.emit_pipeline(
            body,
            grid=(n_rows // _G,),
            in_specs=[pl.BlockSpec((_G, w), index_map=lambda i: (i, 0))],
            out_specs=[pl.BlockSpec((_G * w, emb), index_map=lambda i: (i, 0))],
            core_axis_name=("core", "subcore"),
            dimension_semantics=(pltpu.PARALLEL,),
        )(i_hbm, o_hbm)

    return k(table, idx2d)


def _mask_len_tc(inputs):
    b, l = inputs.shape

    def body(x_ref, m_ref, len_ref):
        x = x_ref[...]
        m = x != _PAD
        m_ref[...] = m
        len_ref[...] = jnp.sum(m.astype(jnp.int32), axis=1, keepdims=True)

    return pl.pallas_call(
        body,
        out_shape=(jax.ShapeDtypeStruct((b, l), jnp.bool_),
                   jax.ShapeDtypeStruct((b, 1), jnp.int32)),
    )(inputs)


def kernel(table, inputs):
    b, l = inputs.shape
    emb = table.shape[1]
    idx2d = inputs.reshape(b * l // _WINDOW, _WINDOW)
    emb_flat = _gather_sc(table, idx2d)
    masks, lengths = _mask_len_tc(inputs)
    return emb_flat.reshape(b, l, emb), lengths.reshape(b), masks


# speedup vs baseline: 9.1996x; 1.0095x over previous
import functools

import jax
import jax.numpy as jnp
from jax.experimental import pallas as pl
from jax.experimental.pallas import tpu as pltpu
from jax.experimental.pallas import tpu_sc as plsc

_PAD = 0
_W = 128
_NB = 6
_K = 4
_NC = 2
_NS = 16


def _gather_sc(table, idx2d):
    n_win, w = idx2d.shape
    emb = table.shape[1]
    n_idx = n_win * w
    nw = _NC * _NS
    n = n_win // nw
    mesh = plsc.VectorSubcoreMesh(core_axis_name="core",
                                  subcore_axis_name="subcore")

    @functools.partial(
        pl.kernel,
        out_type=jax.ShapeDtypeStruct((n_idx, emb), table.dtype),
        mesh=mesh,
        scratch_types=(
            [pltpu.VMEM((n, w), jnp.int32)]
            + [pltpu.VMEM((w, emb), table.dtype) for _ in range(_NB)]
            + [pltpu.SemaphoreType.DMA for _ in range(2 * _NB)]
        ),
    )
    def k(x_hbm, i_hbm, o_hbm, idx_v, *rest):
        bufs = rest[:_NB]
        gsems = rest[_NB:2 * _NB]
        ssems = rest[2 * _NB:]
        wid = jax.lax.axis_index("subcore") * _NC + jax.lax.axis_index("core")
        win0 = wid * n
        row0 = win0 * w

        pltpu.sync_copy(i_hbm.at[pl.ds(win0, n)], idx_v)

        def issue_gather(g, slot):
            pltpu.async_copy(x_hbm.at[idx_v.at[g]], bufs[slot], gsems[slot])

        def wait_gather(slot):
            pltpu.make_async_copy(x_hbm.at[idx_v.at[0]], bufs[slot],
                                  gsems[slot]).wait()

        def issue_store(g, slot):
            pltpu.async_copy(bufs[slot], o_hbm.at[pl.ds(row0 + g * w, w)],
                             ssems[slot])

        def wait_store(slot):
            pltpu.make_async_copy(bufs[slot], o_hbm.at[pl.ds(0, w)],
                                  ssems[slot]).wait()

        for g in range(_K):
            issue_gather(g, g % _NB)

        for g in range(_NB):
            wait_gather(g % _NB)
            issue_store(g, g % _NB)
            if g + _K >= _NB and g + _K < n:
                wait_store((g + _K) % _NB)
            if g + _K < n:
                issue_gather(g + _K, (g + _K) % _NB)

        n_main = ((n - _K - _NB) // _NB) * _NB

        @pl.loop(0, n_main, step=_NB)
        def _(t):
            for b in range(_NB):
                g = _NB + t + b
                wait_gather(b)
                issue_store(g, b)
                wait_store((b + _K) % _NB)
                issue_gather(g + _K, (b + _K) % _NB)

        for g in range(_NB + n_main, n):
            wait_gather(g % _NB)
            issue_store(g, g % _NB)
            if g + _K < n:
                wait_store((g + _K) % _NB)
                issue_gather(g + _K, (g + _K) % _NB)

        for b in range(_NB):
            wait_store(b)

    return k(table, idx2d)


def _mask_len_tc(inputs):
    b, l = inputs.shape

    def body(x_ref, m_ref, len_ref):
        x = x_ref[...]
        m = x != _PAD
        m_ref[...] = m
        len_ref[...] = jnp.sum(m.astype(jnp.int32), axis=1, keepdims=True)

    return pl.pallas_call(
        body,
        out_shape=(jax.ShapeDtypeStruct((b, l), jnp.bool_),
                   jax.ShapeDtypeStruct((b, 1), jnp.int32)),
    )(inputs)


def kernel(table, inputs):
    b, l = inputs.shape
    emb = table.shape[1]
    idx2d = inputs.reshape(b * l // _W, _W)
    emb_flat = _gather_sc(table, idx2d)
    masks, lengths = _mask_len_tc(inputs)
    return emb_flat.reshape(b, l, emb), lengths.reshape(b), masks
